# baseline (speedup 1.0000x reference)
import functools

import jax
import jax.numpy as jnp
from jax import lax
from jax.experimental import pallas as pl
from jax.experimental.pallas import tpu as pltpu
from jax.experimental.pallas import tpu_sc as plsc

_B = 4096
_H = 64
_NC = 2
_NS = 16
_NW = _NC * _NS
_BPW = _B // _NW


@functools.partial(
    pl.kernel,
    mesh=plsc.VectorSubcoreMesh(core_axis_name="c", subcore_axis_name="s"),
    out_type=jax.ShapeDtypeStruct((8, _H, _B), jnp.float32),
    scratch_types=[
        pltpu.VMEM((_BPW,), jnp.int32),
        pltpu.VMEM((_BPW,), jnp.int32),
        pltpu.VMEM((_BPW,), jnp.int32),
        pltpu.VMEM((_BPW,), jnp.int32),
        pltpu.VMEM((_H, _BPW), jnp.float32),
        pltpu.VMEM((_H, _BPW), jnp.float32),
        pltpu.VMEM((_H, _BPW), jnp.float32),
        pltpu.VMEM((_H, _BPW), jnp.float32),
        pltpu.SemaphoreType.DMA,
        pltpu.SemaphoreType.DMA,
    ],
    compiler_params=pltpu.CompilerParams(use_tc_tiling_on_sc=False),
)
def _mdp_embed(ts0, ts1, ts2, ts3, i0, i1, i2, i3, tt, out,
               x0, x1, x2, x3, gb0, gb1, gb2, gb3, lsem, ssem):
    wid = lax.axis_index("s") * _NC + lax.axis_index("c")
    base = wid * _BPW
    states = (ts0, ts1, ts2, ts3)
    idx_hbm = (i0, i1, i2, i3)
    idx = (x0, x1, x2, x3)
    gbufs = (gb0, gb1, gb2, gb3)

    for i in range(4):
        pltpu.sync_copy(idx_hbm[i].at[pl.ds(base, _BPW)], idx[i])

    stores = []
    for i in range(4):
        stores.append(pltpu.async_copy(
            states[i].at[:, pl.ds(base, _BPW)],
            out.at[2 * i, :, pl.ds(base, _BPW)], ssem))

    loads = []
    for i in range(4):
        for j in range(_H):
            loads.append(pltpu.async_copy(
                tt.at[j].at[idx[i]], gbufs[i].at[j], lsem))
    for c in loads:
        c.wait()

    for i in range(4):
        stores.append(pltpu.async_copy(
            gbufs[i], out.at[2 * i + 1, :, pl.ds(base, _BPW)], ssem))
    for c in stores:
        c.wait()


def kernel(s0, a0, s1, a1, s2, a2, s3, a3, table):
    i0 = a0.reshape(-1).astype(jnp.int32)
    i1 = a1.reshape(-1).astype(jnp.int32)
    i2 = a2.reshape(-1).astype(jnp.int32)
    i3 = a3.reshape(-1).astype(jnp.int32)
    out_t = _mdp_embed(s0.T, s1.T, s2.T, s3.T, i0, i1, i2, i3, table.T)
    return out_t.transpose(2, 0, 1)


# speedup vs baseline: 6.0084x; 6.0084x over previous
import functools

import jax
import jax.numpy as jnp
from jax import lax
from jax.experimental import pallas as pl
from jax.experimental.pallas import tpu as pltpu
from jax.experimental.pallas import tpu_sc as plsc

_B = 4096
_H = 64
_V2 = 500000
_NC = 2
_NS = 16
_NW = _NC * _NS
_BPW = _B // _NW


@functools.partial(
    pl.kernel,
    mesh=plsc.VectorSubcoreMesh(core_axis_name="c", subcore_axis_name="s"),
    out_type=jax.ShapeDtypeStruct((_B, 8, _H), jnp.float32),
    scratch_types=[
        pltpu.VMEM((4, _BPW), jnp.int32),
        pltpu.VMEM((_BPW,), jnp.int32),
        pltpu.VMEM((_BPW,), jnp.int32),
        pltpu.VMEM((_BPW,), jnp.int32),
        pltpu.VMEM((_BPW,), jnp.int32),
        pltpu.VMEM((_BPW, 2 * _H), jnp.float32),
        pltpu.VMEM((_BPW, 2 * _H), jnp.float32),
        pltpu.VMEM((_BPW, 2 * _H), jnp.float32),
        pltpu.VMEM((_BPW, 2 * _H), jnp.float32),
        pltpu.VMEM((_BPW, _H), jnp.float32),
        pltpu.SemaphoreType.DMA,
        pltpu.SemaphoreType.DMA,
    ],
    compiler_params=pltpu.CompilerParams(needs_layout_passes=False),
)
def _mdp_embed(s0, s1, s2, s3, i0, i1, i2, i3, P, out,
               idx_v, r0, r1, r2, r3, pb0, pb1, pb2, pb3, obuf,
               lsem, ssem):
    wid = lax.axis_index("s") * _NC + lax.axis_index("c")
    base = wid * _BPW
    states = (s0, s1, s2, s3)
    idx_hbm = (i0, i1, i2, i3)
    rbufs = (r0, r1, r2, r3)
    pbufs = (pb0, pb1, pb2, pb3)

    for i in range(4):
        pltpu.sync_copy(idx_hbm[i].at[pl.ds(base, _BPW)], idx_v.at[i])

    stores = []
    for i in range(4):
        stores.append(pltpu.async_copy(
            states[i].at[pl.ds(base, _BPW)], out.at[pl.ds(base, _BPW), 2 * i], ssem))

    for i in range(4):
        for k in range(_BPW // 16):
            v = idx_v[i, pl.ds(k * 16, 16)]
            rbufs[i][pl.ds(k * 16, 16)] = lax.shift_right_logical(v, 1)

    loads = []
    for i in range(4):
        loads.append(pltpu.async_copy(P.at[rbufs[i]], pbufs[i], lsem))
    for c in loads:
        c.wait()

    lane = lax.iota(jnp.int32, 16)
    for i in range(4):
        for k in range(_BPW // 16):
            v = idx_v[i, pl.ds(k * 16, 16)]
            half = lax.bitwise_and(v, 1) * _H
            rows = k * 16 + lane
            for f in range(_H):
                fv = jnp.full((16,), f, jnp.int32)
                vals = plsc.load_gather(pbufs[i], [rows, half + f])
                plsc.store_scatter(obuf, [rows, fv], vals)
        pltpu.sync_copy(obuf, out.at[pl.ds(base, _BPW), 2 * i + 1])
    for c in stores:
        c.wait()


def kernel(s0, a0, s1, a1, s2, a2, s3, a3, table):
    i0 = a0.reshape(-1).astype(jnp.int32)
    i1 = a1.reshape(-1).astype(jnp.int32)
    i2 = a2.reshape(-1).astype(jnp.int32)
    i3 = a3.reshape(-1).astype(jnp.int32)
    P = table[:2 * _V2].reshape(_V2, 2 * _H)
    return _mdp_embed(s0, s1, s2, s3, i0, i1, i2, i3, P)


# speedup vs baseline: 8.4369x; 1.4042x over previous
import functools

import jax
import jax.numpy as jnp
from jax import lax
from jax.experimental import pallas as pl
from jax.experimental.pallas import tpu as pltpu
from jax.experimental.pallas import tpu_sc as plsc

_B = 4096
_H = 64
_NC = 2
_NS = 16
_NW = _NC * _NS
_BPW = _B // _NW


@functools.partial(
    pl.kernel,
    mesh=plsc.VectorSubcoreMesh(core_axis_name="c", subcore_axis_name="s"),
    out_type=jax.ShapeDtypeStruct((_B, 8, _H), jnp.float32),
    scratch_types=[
        pltpu.VMEM((4, _BPW), jnp.int32),
        pltpu.VMEM((_BPW, _H), jnp.float32),
        pltpu.VMEM((_BPW, _H), jnp.float32),
        pltpu.VMEM((_BPW, _H), jnp.float32),
        pltpu.VMEM((_BPW, _H), jnp.float32),
        pltpu.SemaphoreType.DMA,
        pltpu.SemaphoreType.DMA,
    ],
)
def _mdp_embed(s0, s1, s2, s3, i0, i1, i2, i3, table, out,
               idx_v, gb0, gb1, gb2, gb3, lsem, ssem):
    wid = lax.axis_index("s") * _NC + lax.axis_index("c")
    base = wid * _BPW
    states = (s0, s1, s2, s3)
    idx_hbm = (i0, i1, i2, i3)
    gbufs = (gb0, gb1, gb2, gb3)

    for i in range(4):
        pltpu.sync_copy(idx_hbm[i].at[pl.ds(base, _BPW)], idx_v.at[i])

    stores = []
    for i in range(4):
        stores.append(pltpu.async_copy(
            states[i].at[pl.ds(base, _BPW)], out.at[pl.ds(base, _BPW), 2 * i], ssem))

    def vec_body(k, _):
        for i in range(4):
            v = idx_v[i, pl.ds(k * 16, 16)]
            for j in range(16):
                row = v[j]
                pltpu.async_copy(table.at[pl.ds(row, 1), :],
                                 gbufs[i].at[pl.ds(k * 16 + j, 1), :], lsem)
        return 0

    lax.fori_loop(0, _BPW // 16, vec_body, 0)

    for i in range(4):
        pltpu.make_async_copy(table.at[pl.ds(0, _BPW), :], gbufs[i], lsem).wait()

    for i in range(4):
        stores.append(pltpu.async_copy(gbufs[i], out.at[pl.ds(base, _BPW), 2 * i + 1], ssem))
    for c in stores:
        c.wait()


def kernel(s0, a0, s1, a1, s2, a2, s3, a3, table):
    i0 = a0.reshape(-1).astype(jnp.int32)
    i1 = a1.reshape(-1).astype(jnp.int32)
    i2 = a2.reshape(-1).astype(jnp.int32)
    i3 = a3.reshape(-1).astype(jnp.int32)
    return _mdp_embed(s0, s1, s2, s3, i0, i1, i2, i3, table)


# speedup vs baseline: 9.1192x; 1.0809x over previous
import functools

import jax
import jax.numpy as jnp
from jax import lax
from jax.experimental import pallas as pl
from jax.experimental.pallas import tpu as pltpu
from jax.experimental.pallas import tpu_sc as plsc

_B = 4096
_H = 64
_W = 2 * _H
_NC = 2
_NS = 16
_NW = _NC * _NS
_BPW = _B // _NW


@functools.partial(
    pl.kernel,
    mesh=plsc.VectorSubcoreMesh(core_axis_name="c", subcore_axis_name="s"),
    out_type=jax.ShapeDtypeStruct((4, _B, _W), jnp.float32),
    scratch_types=[
        pltpu.VMEM((_BPW,), jnp.int32),
        pltpu.VMEM((_BPW,), jnp.int32),
        pltpu.VMEM((_BPW,), jnp.int32),
        pltpu.VMEM((_BPW,), jnp.int32),
        pltpu.VMEM((_BPW, _W), jnp.float32),
        pltpu.VMEM((_BPW, _W), jnp.float32),
        pltpu.VMEM((_BPW, _W), jnp.float32),
        pltpu.VMEM((_BPW, _W), jnp.float32),
        pltpu.SemaphoreType.DMA,
        pltpu.SemaphoreType.DMA,
    ],
)
def _mdp_gather(i0, i1, i2, i3, P, out,
                x0, x1, x2, x3, gb0, gb1, gb2, gb3, lsem, ssem):
    wid = lax.axis_index("s") * _NC + lax.axis_index("c")
    base = wid * _BPW
    idx_hbm = (i0, i1, i2, i3)
    idx = (x0, x1, x2, x3)
    gbufs = (gb0, gb1, gb2, gb3)

    for i in range(4):
        pltpu.sync_copy(idx_hbm[i].at[pl.ds(base, _BPW)], idx[i])

    loads = []
    for i in range(4):
        loads.append(pltpu.async_copy(P.at[idx[i]], gbufs[i], lsem))
    for c in loads:
        c.wait()

    stores = []
    for i in range(4):
        stores.append(pltpu.async_copy(
            gbufs[i], out.at[i, pl.ds(base, _BPW), :], ssem))
    for c in stores:
        c.wait()


def kernel(s0, a0, s1, a1, s2, a2, s3, a3, table):
    i0 = a0.reshape(-1).astype(jnp.int32)
    i1 = a1.reshape(-1).astype(jnp.int32)
    i2 = a2.reshape(-1).astype(jnp.int32)
    i3 = a3.reshape(-1).astype(jnp.int32)
    P = jnp.pad(table, ((0, 0), (0, _W - _H)))
    G = _mdp_gather(i0, i1, i2, i3, P)
    pieces = []
    for i, s in enumerate((s0, s1, s2, s3)):
        pieces.append(s[:, None, :])
        pieces.append(G[i, :, None, :_H])
    return jnp.concatenate(pieces, axis=1)
